# baseline (speedup 1.0000x reference)
import functools
import math

import jax
import jax.numpy as jnp
from jax.experimental import pallas as pl
from jax.experimental.pallas import tpu as pltpu

_EPS = 1e-10
_NEG_LOG2EPS = -math.log2(1e-10)
_LOG2E = math.log2(math.e)


def _lambda_loss_block(p_ref, t_ref, pt_ref, out_ref):
    R, _, N = p_ref.shape
    one = jnp.float32(1.0)
    zero = jnp.float32(0.0)

    pb = p_ref[...]
    tb = t_ref[...]
    pa = jnp.transpose(pt_ref[...], (2, 1, 0))

    ia = jax.lax.broadcasted_iota(jnp.int32, (1, N, N), 1)
    ib = jax.lax.broadcasted_iota(jnp.int32, (1, N, N), 2)
    tie_blt = jnp.where(ib < ia, one, zero)

    cmp = jnp.where(pb > pa, one, jnp.where(pb == pa, tie_blt, zero))
    ones_col = jnp.ones((N, 1), jnp.float32)
    rank_a = jax.lax.dot_general(cmp, ones_col, (((2,), (0,)), ((), ())),
                                 preferred_element_type=jnp.float32)
    inv_d_a = one / jnp.log2(rank_a + 2.0)
    inv_d_b = jnp.transpose(inv_d_a, (0, 2, 1))

    pos = jax.lax.broadcasted_iota(jnp.int32, (1, 1, N), 2).astype(jnp.float32)
    inv_disc = one / jnp.log2(pos + 2.0)
    max_dcg = jnp.full((R, 1, 1), _EPS, jnp.float32)
    for v in (1, 2, 3, 4):
        cnt = jnp.sum(jnp.where(tb >= v, one, zero), axis=2, keepdims=True)
        s_v = jnp.sum(jnp.where(pos < cnt, inv_disc, zero), axis=2,
                      keepdims=True)
        max_dcg = max_dcg + (2.0 ** (v - 1)) * s_v
    inv_max_dcg = one / max_dcg
    gain_b = (jnp.exp2(tb) - one) * inv_max_dcg
    gain_a = jnp.transpose(gain_b, (0, 2, 1))

    dg = gain_a - gain_b
    w = jnp.abs(inv_d_a - inv_d_b) * dg
    nls = jnp.log2(one + jnp.exp2(pb - pa))
    nls = jnp.minimum(nls, _NEG_LOG2EPS)
    contrib = jnp.maximum(w * nls, zero)
    s1 = jax.lax.dot_general(contrib, ones_col, (((2,), (0,)), ((), ())),
                             preferred_element_type=jnp.float32)
    s2 = jax.lax.dot_general(s1, ones_col, (((1,), (0,)), ((), ())),
                             preferred_element_type=jnp.float32)
    out_ref[...] = jnp.sum(s2, axis=(0, 1, 2), keepdims=True)


@jax.jit
def kernel(pred_scores, labels):
    B, N = pred_scores.shape
    R = 32
    grid = B // R
    t = labels.astype(jnp.float32)
    ps = pred_scores * jnp.float32(_LOG2E)
    p3 = ps.reshape(B, 1, N)
    t3 = t.reshape(B, 1, N)
    pt = ps.reshape(grid, R, N).transpose(0, 2, 1)

    partials = pl.pallas_call(
        _lambda_loss_block,
        grid=(grid,),
        in_specs=[
            pl.BlockSpec((R, 1, N), lambda i: (i, 0, 0)),
            pl.BlockSpec((R, 1, N), lambda i: (i, 0, 0)),
            pl.BlockSpec((1, N, R), lambda i: (i, 0, 0)),
        ],
        out_specs=pl.BlockSpec((1, 1, 1), lambda i: (i, 0, 0)),
        out_shape=jax.ShapeDtypeStruct((grid, 1, 1), jnp.float32),
        compiler_params=pltpu.CompilerParams(
            dimension_semantics=("arbitrary",),
        ),
    )(p3, t3, pt)
    return jnp.sum(partials).reshape(())


# speedup vs baseline: 1.0117x; 1.0117x over previous
import functools
import math

import jax
import jax.numpy as jnp
from jax.experimental import pallas as pl
from jax.experimental.pallas import tpu as pltpu

_EPS = 1e-10
_NEG_LOG2EPS = -math.log2(1e-10)
_LOG2E = math.log2(math.e)
_CHUNK = 40


def _lambda_loss_block(p_ref, t_ref, pt_ref, tt_ref, out_ref):
    R, _, N = p_ref.shape
    one = jnp.float32(1.0)
    zero = jnp.float32(0.0)

    pb = p_ref[...]
    tb = t_ref[...]
    pa = jnp.transpose(pt_ref[...], (2, 1, 0))
    ta = jnp.transpose(tt_ref[...], (2, 1, 0))

    ia = jax.lax.broadcasted_iota(jnp.int32, (1, N, N), 1)
    ib = jax.lax.broadcasted_iota(jnp.int32, (1, N, N), 2)
    tie_blt = jnp.where(ib < ia, one, zero)

    ones_col = jnp.ones((N, 1), jnp.float32)
    rank_chunks = []
    for c in range(0, N, _CHUNK):
        pa_c = pa[:, c:c + _CHUNK, :]
        tie_c = tie_blt[:, c:c + _CHUNK, :]
        cmp_c = jnp.where(pb > pa_c, one, jnp.where(pb == pa_c, tie_c, zero))
        rank_chunks.append(
            jax.lax.dot_general(cmp_c, ones_col, (((2,), (0,)), ((), ())),
                                preferred_element_type=jnp.float32))
    rank_a = jnp.concatenate(rank_chunks, axis=1)
    inv_d_a = one / jnp.log2(rank_a + 2.0)
    inv_d_b = jnp.transpose(inv_d_a, (0, 2, 1))

    pos = jax.lax.broadcasted_iota(jnp.int32, (1, 1, N), 2).astype(jnp.float32)
    inv_disc = one / jnp.log2(pos + 2.0)
    max_dcg = jnp.full((R, 1, 1), _EPS, jnp.float32)
    for v in (1, 2, 3, 4):
        cnt = jnp.sum(jnp.where(tb >= v, one, zero), axis=2, keepdims=True)
        s_v = jnp.sum(jnp.where(pos < cnt, inv_disc, zero), axis=2,
                      keepdims=True)
        max_dcg = max_dcg + (2.0 ** (v - 1)) * s_v
    inv_max_dcg = one / max_dcg
    gain_a = (jnp.exp2(ta) - one) * inv_max_dcg
    gain_b = (jnp.exp2(tb) - one) * inv_max_dcg

    s1_chunks = []
    for c in range(0, N, _CHUNK):
        pa_c = pa[:, c:c + _CHUNK, :]
        dg_c = gain_a[:, c:c + _CHUNK, :] - gain_b
        w_c = jnp.abs(inv_d_a[:, c:c + _CHUNK, :] - inv_d_b) * dg_c
        nls_c = jnp.log2(one + jnp.exp2(pb - pa_c))
        nls_c = jnp.minimum(nls_c, _NEG_LOG2EPS)
        contrib_c = jnp.maximum(w_c * nls_c, zero)
        s1_chunks.append(
            jax.lax.dot_general(contrib_c, ones_col, (((2,), (0,)), ((), ())),
                                preferred_element_type=jnp.float32))
    s1 = jnp.concatenate(s1_chunks, axis=1)
    s2 = jax.lax.dot_general(s1, ones_col, (((1,), (0,)), ((), ())),
                             preferred_element_type=jnp.float32)
    out_ref[...] = jnp.sum(s2, axis=(0, 1, 2), keepdims=True)


@jax.jit
def kernel(pred_scores, labels):
    B, N = pred_scores.shape
    R = 32
    grid = B // R
    t = labels.astype(jnp.float32)
    ps = pred_scores * jnp.float32(_LOG2E)
    p3 = ps.reshape(B, 1, N)
    t3 = t.reshape(B, 1, N)
    pt = ps.reshape(grid, R, N).transpose(0, 2, 1)
    tt = t.reshape(grid, R, N).transpose(0, 2, 1)

    partials = pl.pallas_call(
        _lambda_loss_block,
        grid=(grid,),
        in_specs=[
            pl.BlockSpec((R, 1, N), lambda i: (i, 0, 0)),
            pl.BlockSpec((R, 1, N), lambda i: (i, 0, 0)),
            pl.BlockSpec((1, N, R), lambda i: (i, 0, 0)),
            pl.BlockSpec((1, N, R), lambda i: (i, 0, 0)),
        ],
        out_specs=pl.BlockSpec((1, 1, 1), lambda i: (i, 0, 0)),
        out_shape=jax.ShapeDtypeStruct((grid, 1, 1), jnp.float32),
        compiler_params=pltpu.CompilerParams(
            dimension_semantics=("arbitrary",),
        ),
    )(p3, t3, pt, tt)
    return jnp.sum(partials).reshape(())


# speedup vs baseline: 1.0573x; 1.0451x over previous
import functools
import math

import jax
import jax.numpy as jnp
from jax.experimental import pallas as pl
from jax.experimental.pallas import tpu as pltpu

_EPS = 1e-10
_NEG_LOG2EPS = -math.log2(1e-10)
_LOG2E = math.log2(math.e)


def _lambda_loss_block(p_ref, t_ref, pt_ref, tt_ref, out_ref):
    R, _, N = p_ref.shape
    one = jnp.float32(1.0)
    zero = jnp.float32(0.0)

    pb = p_ref[...]
    tb = t_ref[...]
    pa = jnp.transpose(pt_ref[...], (2, 1, 0))
    ta = jnp.transpose(tt_ref[...], (2, 1, 0))

    ia = jax.lax.broadcasted_iota(jnp.int32, (1, N, N), 1)
    ib = jax.lax.broadcasted_iota(jnp.int32, (1, N, N), 2)
    tie_blt = jnp.where(ib < ia, one, zero)

    cmp = jnp.where(pb > pa, one, jnp.where(pb == pa, tie_blt, zero))
    ones_col = jnp.ones((N, 1), jnp.float32)
    rank_a = jax.lax.dot_general(cmp, ones_col, (((2,), (0,)), ((), ())),
                                 preferred_element_type=jnp.float32)
    inv_d_a = one / jnp.log2(rank_a + 2.0)
    inv_d_b = jnp.transpose(inv_d_a, (0, 2, 1))

    pos = jax.lax.broadcasted_iota(jnp.int32, (1, 1, N), 2).astype(jnp.float32)
    inv_disc = one / jnp.log2(pos + 2.0)
    max_dcg = jnp.full((R, 1, 1), _EPS, jnp.float32)
    for v in (1, 2, 3, 4):
        cnt = jnp.sum(jnp.where(tb >= v, one, zero), axis=2, keepdims=True)
        s_v = jnp.sum(jnp.where(pos < cnt, inv_disc, zero), axis=2,
                      keepdims=True)
        max_dcg = max_dcg + (2.0 ** (v - 1)) * s_v
    inv_max_dcg = one / max_dcg
    gain_a = (jnp.exp2(ta) - one) * inv_max_dcg
    gain_b = (jnp.exp2(tb) - one) * inv_max_dcg

    dg = gain_a - gain_b
    w = jnp.abs(inv_d_a - inv_d_b) * dg
    nls = jnp.log2(one + jnp.exp2(pb - pa))
    nls = jnp.minimum(nls, _NEG_LOG2EPS)
    contrib = jnp.maximum(w * nls, zero)
    s1 = jax.lax.dot_general(contrib, ones_col, (((2,), (0,)), ((), ())),
                             preferred_element_type=jnp.float32)
    s2 = jax.lax.dot_general(s1, ones_col, (((1,), (0,)), ((), ())),
                             preferred_element_type=jnp.float32)
    out_ref[...] = jnp.sum(s2, axis=(0, 1, 2), keepdims=True)


@jax.jit
def kernel(pred_scores, labels):
    B, N = pred_scores.shape
    R = 64
    grid = B // R
    t = labels.astype(jnp.float32)
    ps = pred_scores * jnp.float32(_LOG2E)
    p3 = ps.reshape(B, 1, N)
    t3 = t.reshape(B, 1, N)
    pt = ps.reshape(grid, R, N).transpose(0, 2, 1)
    tt = t.reshape(grid, R, N).transpose(0, 2, 1)

    partials = pl.pallas_call(
        _lambda_loss_block,
        grid=(grid,),
        in_specs=[
            pl.BlockSpec((R, 1, N), lambda i: (i, 0, 0)),
            pl.BlockSpec((R, 1, N), lambda i: (i, 0, 0)),
            pl.BlockSpec((1, N, R), lambda i: (i, 0, 0)),
            pl.BlockSpec((1, N, R), lambda i: (i, 0, 0)),
        ],
        out_specs=pl.BlockSpec((1, 1, 1), lambda i: (i, 0, 0)),
        out_shape=jax.ShapeDtypeStruct((grid, 1, 1), jnp.float32),
        compiler_params=pltpu.CompilerParams(
            dimension_semantics=("arbitrary",),
        ),
    )(p3, t3, pt, tt)
    return jnp.sum(partials).reshape(())
